# baseline (speedup 1.0000x reference)
import functools

import jax
import jax.numpy as jnp
import numpy as np
from jax import lax
from jax.experimental import pallas as pl
from jax.experimental.pallas import tpu as pltpu
from jax.experimental.pallas import tpu_sc as plsc

N_NODES = 10000
N_EDGES = 320000
IN_DIM = 128
OUT_DIM = 16
N_HEADS = 8
ROW = OUT_DIM * N_HEADS
ZROW = ROW + 16

NC = 2
NS = 16
NW = NC * NS
EPW = N_EDGES // NW
CHUNK = 40
NCHUNK = EPW // CHUNK
RPT = N_NODES // NS


def _qkv_body(x_ref, wq_ref, wk_ref, wv_ref, q_ref, k_ref, v_ref):
    xb = x_ref[...]
    q_ref[...] = jnp.dot(xb, wq_ref[...], preferred_element_type=jnp.float32)
    k_ref[...] = jnp.dot(xb, wk_ref[...], preferred_element_type=jnp.float32)
    v_ref[...] = jnp.dot(xb, wv_ref[...], preferred_element_type=jnp.float32)


def _qkv(x, Wq, Wk, Wv):
    blk = 2000
    grid = (N_NODES // blk,)
    out = jax.ShapeDtypeStruct((N_NODES, ROW), jnp.float32)
    return pl.pallas_call(
        _qkv_body,
        grid=grid,
        in_specs=[
            pl.BlockSpec((blk, IN_DIM), lambda i: (i, 0)),
            pl.BlockSpec((IN_DIM, ROW), lambda i: (0, 0)),
            pl.BlockSpec((IN_DIM, ROW), lambda i: (0, 0)),
            pl.BlockSpec((IN_DIM, ROW), lambda i: (0, 0)),
        ],
        out_specs=[
            pl.BlockSpec((blk, ROW), lambda i: (i, 0)),
            pl.BlockSpec((blk, ROW), lambda i: (i, 0)),
            pl.BlockSpec((blk, ROW), lambda i: (i, 0)),
        ],
        out_shape=[out, out, out],
    )(x, Wq, Wk, Wv)


_sc_mesh = plsc.VectorSubcoreMesh(core_axis_name="c", subcore_axis_name="s")


@functools.partial(
    pl.kernel,
    mesh=_sc_mesh,
    compiler_params=pltpu.CompilerParams(use_tc_tiling_on_sc=False,
                                         needs_layout_passes=False),
    out_type=(jax.ShapeDtypeStruct((NC, N_NODES, ROW), jnp.float32),
              jax.ShapeDtypeStruct((NC, N_NODES, 16), jnp.float32)),
    scratch_types=[
        pltpu.VMEM((4, CHUNK), jnp.int32),
        pltpu.VMEM((4, CHUNK), jnp.int32),
        pltpu.VMEM((2, CHUNK, ROW), jnp.float32),
        pltpu.VMEM((2, CHUNK, ROW), jnp.float32),
        pltpu.VMEM((2, CHUNK, ROW), jnp.float32),
        pltpu.VMEM((2, CHUNK, 16), jnp.float32),
        pltpu.VMEM_SHARED((N_NODES, ROW), jnp.float32),
        pltpu.VMEM_SHARED((N_NODES, 16), jnp.float32),
        pltpu.SemaphoreType.DMA,
        pltpu.SemaphoreType.DMA,
        pltpu.SemaphoreType.DMA,
        pltpu.SemaphoreType.DMA,
        pltpu.SemaphoreType.DMA,
        pltpu.SemaphoreType.DMA,
        pltpu.SemaphoreType.DMA,
    ],
)
def _sc_attn(k_hbm, q_hbm, v_hbm, edge_hbm, outwv_hbm, outz_hbm,
             srcv, dstv, kr, qr, vr, zmsg, accwv, accz,
             sem1, sem2, sem3, semi0, semi1, semw, semz):
    cid = lax.axis_index("c")
    sid = lax.axis_index("s")
    wid = sid * NC + cid

    zeros16 = jnp.zeros((16,), jnp.float32)

    def zero_row(r, carry):
        for j in range(ROW // 16):
            vr[0, r, pl.ds(j * 16, 16)] = zeros16
        zmsg[0, r, pl.ds(0, 16)] = zeros16
        return carry

    lax.fori_loop(0, CHUNK, zero_row, 0)

    def zero_chunk(j, carry):
        c = sid + j * NS

        @pl.when(c < N_NODES // CHUNK)
        def _():
            pltpu.sync_copy(vr.at[0], accwv.at[pl.ds(c * CHUNK, CHUNK)])
            pltpu.sync_copy(zmsg.at[0], accz.at[pl.ds(c * CHUNK, CHUNK)])

        return carry

    lax.fori_loop(0, (N_NODES // CHUNK + NS - 1) // NS, zero_chunk, 0)
    plsc.subcore_barrier()

    lanes = lax.iota(jnp.int32, 16)
    last = jnp.full((16,), 15, jnp.int32)

    def fetch_idx(i, bi, semi):
        base = wid * EPW + i * CHUNK
        pltpu.async_copy(edge_hbm.at[0, pl.ds(base, CHUNK)], srcv.at[bi], semi)
        pltpu.async_copy(edge_hbm.at[1, pl.ds(base, CHUNK)], dstv.at[bi], semi)

    def wait_idx(i, bi, semi):
        base = wid * EPW + i * CHUNK
        pltpu.make_async_copy(edge_hbm.at[0, pl.ds(base, CHUNK)],
                              srcv.at[bi], semi).wait()
        pltpu.make_async_copy(edge_hbm.at[1, pl.ds(base, CHUNK)],
                              dstv.at[bi], semi).wait()

    def fetch_rows(bi, b):
        pltpu.async_copy(k_hbm.at[srcv.at[bi]], kr.at[b], sem1)
        pltpu.async_copy(q_hbm.at[dstv.at[bi]], qr.at[b], sem2)
        pltpu.async_copy(v_hbm.at[srcv.at[bi]], vr.at[b], sem3)

    def drain_rows(bi, b):
        pltpu.make_async_copy(k_hbm.at[srcv.at[bi]], kr.at[b], sem1).wait()
        pltpu.make_async_copy(q_hbm.at[dstv.at[bi]], qr.at[b], sem2).wait()
        pltpu.make_async_copy(v_hbm.at[srcv.at[bi]], vr.at[b], sem3).wait()

    def start_scatter(bi, b):
        pltpu.async_copy(vr.at[b], accwv.at[dstv.at[bi]], semw, add=True)
        pltpu.async_copy(zmsg.at[b], accz.at[dstv.at[bi]], semz, add=True)

    def wait_scatter(bi, b):
        pltpu.make_async_copy(vr.at[b], accwv.at[dstv.at[bi]], semw).wait()
        pltpu.make_async_copy(zmsg.at[b], accz.at[dstv.at[bi]], semz).wait()

    fetch_idx(0, 0, semi0)
    fetch_idx(1, 1, semi1)
    wait_idx(0, 0, semi0)
    fetch_rows(0, 0)

    def chunk_body(i, carry):
        b = lax.rem(i, 2)
        bi = lax.rem(i, 4)
        bi1 = lax.rem(i + 1, 4)

        @pl.when(jnp.logical_and(i + 2 < NCHUNK, lax.rem(i, 2) == 0))
        def _():
            fetch_idx(i + 2, lax.rem(i + 2, 4), semi0)

        @pl.when(jnp.logical_and(i + 2 < NCHUNK, lax.rem(i, 2) == 1))
        def _():
            fetch_idx(i + 2, lax.rem(i + 2, 4), semi1)

        @pl.when(i + 1 < NCHUNK)
        def _():
            @pl.when(lax.rem(i + 1, 2) == 0)
            def _():
                wait_idx(i + 1, bi1, semi0)

            @pl.when(lax.rem(i + 1, 2) == 1)
            def _():
                wait_idx(i + 1, bi1, semi1)

            @pl.when(i >= 1)
            def _():
                wait_scatter(lax.rem(i + 3, 4), 1 - b)

            fetch_rows(bi1, 1 - b)

        drain_rows(bi, b)

        @plsc.parallel_loop(0, CHUNK, unroll=8)
        def edge_body(e):
            zvec = jnp.zeros((16,), jnp.float32)
            for h in range(N_HEADS):
                kv = kr[b, e, pl.ds(h * 16, 16)]
                qv = qr[b, e, pl.ds(h * 16, 16)]
                p = plsc.cumsum(kv * qv)
                s = jnp.clip(p.at[last].get(mode="promise_in_bounds"),
                             -5.0, 5.0)
                sv = jnp.exp(s)
                vr[b, e, pl.ds(h * 16, 16)] = vr[b, e, pl.ds(h * 16, 16)] * sv
                zvec = jnp.where(lanes == h, sv, zvec)
            zmsg[b, e, pl.ds(0, 16)] = zvec

        start_scatter(bi, b)
        return carry

    lax.fori_loop(0, NCHUNK, chunk_body, 0)
    wait_scatter((NCHUNK - 1) % 4, (NCHUNK - 1) % 2)
    plsc.subcore_barrier()

    pltpu.sync_copy(accwv.at[pl.ds(sid * RPT, RPT)],
                    outwv_hbm.at[cid, pl.ds(sid * RPT, RPT)])
    pltpu.sync_copy(accz.at[pl.ds(sid * RPT, RPT)],
                    outz_hbm.at[cid, pl.ds(sid * RPT, RPT)])


def _combine_body(pwv_ref, pz_ref, b_ref, o_ref):
    wv = pwv_ref[0] + pwv_ref[1]
    z = pz_ref[0, :, :N_HEADS] + pz_ref[1, :, :N_HEADS]
    zb = jnp.dot(z, b_ref[...], preferred_element_type=jnp.float32)
    o_ref[...] = wv / (zb + 1e-6)


def _combine(pwv, pz, bmat):
    return pl.pallas_call(
        _combine_body,
        out_shape=jax.ShapeDtypeStruct((N_NODES, ROW), jnp.float32),
    )(pwv, pz, bmat)


_BMAT = np.zeros((N_HEADS, ROW), dtype=np.float32)
for _h in range(N_HEADS):
    _BMAT[_h, _h * OUT_DIM:(_h + 1) * OUT_DIM] = 1.0
_BMAT.setflags(write=False)


def kernel(x, edge_index, Wq, Wk, Wv):
    q, k, v = _qkv(x, Wq, Wk * (1.0 / np.sqrt(OUT_DIM)), Wv)
    pwv, pz = _sc_attn(k, q, v, edge_index)
    out = _combine(pwv, pz, jnp.asarray(_BMAT))
    return out.reshape(N_NODES, N_HEADS, OUT_DIM)


# speedup vs baseline: 4.5714x; 4.5714x over previous
import functools

import jax
import jax.numpy as jnp
import numpy as np
from jax import lax
from jax.experimental import pallas as pl
from jax.experimental.pallas import tpu as pltpu
from jax.experimental.pallas import tpu_sc as plsc

N_NODES = 10000
N_EDGES = 320000
IN_DIM = 128
OUT_DIM = 16
N_HEADS = 8
ROW = OUT_DIM * N_HEADS
ZROW = ROW + 16

NC = 2
NS = 16
NW = NC * NS
EPW = N_EDGES // NW
CHUNK = 40
NCHUNK = EPW // CHUNK
RPT = N_NODES // NS


def _qkv_body(x_ref, wq_ref, wk_ref, wv_ref, q_ref, k_ref, v_ref):
    xb = x_ref[...]
    q_ref[...] = jnp.dot(xb, wq_ref[...], preferred_element_type=jnp.float32)
    k_ref[...] = jnp.dot(xb, wk_ref[...], preferred_element_type=jnp.float32)
    v_ref[...] = jnp.dot(xb, wv_ref[...], preferred_element_type=jnp.float32)


def _qkv(x, Wq, Wk, Wv):
    blk = 2000
    grid = (N_NODES // blk,)
    out = jax.ShapeDtypeStruct((N_NODES, ROW), jnp.float32)
    return pl.pallas_call(
        _qkv_body,
        grid=grid,
        in_specs=[
            pl.BlockSpec((blk, IN_DIM), lambda i: (i, 0)),
            pl.BlockSpec((IN_DIM, ROW), lambda i: (0, 0)),
            pl.BlockSpec((IN_DIM, ROW), lambda i: (0, 0)),
            pl.BlockSpec((IN_DIM, ROW), lambda i: (0, 0)),
        ],
        out_specs=[
            pl.BlockSpec((blk, ROW), lambda i: (i, 0)),
            pl.BlockSpec((blk, ROW), lambda i: (i, 0)),
            pl.BlockSpec((blk, ROW), lambda i: (i, 0)),
        ],
        out_shape=[out, out, out],
    )(x, Wq, Wk, Wv)


_sc_mesh = plsc.VectorSubcoreMesh(core_axis_name="c", subcore_axis_name="s")


@functools.partial(
    pl.kernel,
    mesh=_sc_mesh,
    compiler_params=pltpu.CompilerParams(use_tc_tiling_on_sc=False,
                                         needs_layout_passes=False),
    out_type=(jax.ShapeDtypeStruct((NC, N_NODES, ROW), jnp.float32),
              jax.ShapeDtypeStruct((NC, N_NODES, 16), jnp.float32)),
    scratch_types=[
        pltpu.VMEM((4, CHUNK), jnp.int32),
        pltpu.VMEM((4, CHUNK), jnp.int32),
        pltpu.VMEM((2, CHUNK, ROW), jnp.float32),
        pltpu.VMEM((2, CHUNK, ROW), jnp.float32),
        pltpu.VMEM((2, CHUNK, ROW), jnp.float32),
        pltpu.VMEM((2, CHUNK, 16), jnp.float32),
        pltpu.VMEM_SHARED((N_NODES, ROW), jnp.float32),
        pltpu.VMEM_SHARED((N_NODES, 16), jnp.float32),
        pltpu.SemaphoreType.DMA,
        pltpu.SemaphoreType.DMA,
        pltpu.SemaphoreType.DMA,
        pltpu.SemaphoreType.DMA,
        pltpu.SemaphoreType.DMA,
        pltpu.SemaphoreType.DMA,
        pltpu.SemaphoreType.DMA,
    ],
)
def _sc_attn(k_hbm, q_hbm, v_hbm, edge_hbm, outwv_hbm, outz_hbm,
             srcv, dstv, kr, qr, vr, zmsg, accwv, accz,
             sem1, sem2, sem3, semi0, semi1, semw, semz):
    cid = lax.axis_index("c")
    sid = lax.axis_index("s")
    wid = sid * NC + cid

    zeros16 = jnp.zeros((16,), jnp.float32)

    def zero_row(r, carry):
        for j in range(ROW // 16):
            vr[0, r, pl.ds(j * 16, 16)] = zeros16
        zmsg[0, r, pl.ds(0, 16)] = zeros16
        return carry

    lax.fori_loop(0, CHUNK, zero_row, 0)

    def zero_chunk(j, carry):
        c = sid + j * NS

        @pl.when(c < N_NODES // CHUNK)
        def _():
            pltpu.sync_copy(vr.at[0], accwv.at[pl.ds(c * CHUNK, CHUNK)])
            pltpu.sync_copy(zmsg.at[0], accz.at[pl.ds(c * CHUNK, CHUNK)])

        return carry

    lax.fori_loop(0, (N_NODES // CHUNK + NS - 1) // NS, zero_chunk, 0)
    plsc.subcore_barrier()

    lanes = lax.iota(jnp.int32, 16)
    last = jnp.full((16,), 15, jnp.int32)

    def fetch_idx(i, bi, semi):
        base = wid * EPW + i * CHUNK
        pltpu.async_copy(edge_hbm.at[0, pl.ds(base, CHUNK)], srcv.at[bi], semi)
        pltpu.async_copy(edge_hbm.at[1, pl.ds(base, CHUNK)], dstv.at[bi], semi)

    def wait_idx(i, bi, semi):
        base = wid * EPW + i * CHUNK
        pltpu.make_async_copy(edge_hbm.at[0, pl.ds(base, CHUNK)],
                              srcv.at[bi], semi).wait()
        pltpu.make_async_copy(edge_hbm.at[1, pl.ds(base, CHUNK)],
                              dstv.at[bi], semi).wait()

    def fetch_rows(bi, b):
        pltpu.async_copy(k_hbm.at[srcv.at[bi]], kr.at[b], sem1)
        pltpu.async_copy(q_hbm.at[dstv.at[bi]], qr.at[b], sem2)
        pltpu.async_copy(v_hbm.at[srcv.at[bi]], vr.at[b], sem3)

    def drain_rows(bi, b):
        pltpu.make_async_copy(k_hbm.at[srcv.at[bi]], kr.at[b], sem1).wait()
        pltpu.make_async_copy(q_hbm.at[dstv.at[bi]], qr.at[b], sem2).wait()
        pltpu.make_async_copy(v_hbm.at[srcv.at[bi]], vr.at[b], sem3).wait()

    def start_scatter(bi, b):
        pltpu.async_copy(vr.at[b], accwv.at[dstv.at[bi]], semw, add=True)
        pltpu.async_copy(zmsg.at[b], accz.at[dstv.at[bi]], semz, add=True)

    def wait_scatter(bi, b):
        pltpu.make_async_copy(vr.at[b], accwv.at[dstv.at[bi]], semw).wait()
        pltpu.make_async_copy(zmsg.at[b], accz.at[dstv.at[bi]], semz).wait()

    fetch_idx(0, 0, semi0)
    fetch_idx(1, 1, semi1)
    wait_idx(0, 0, semi0)
    fetch_rows(0, 0)

    def chunk_body(i, carry):
        b = lax.rem(i, 2)
        bi = lax.rem(i, 4)
        bi1 = lax.rem(i + 1, 4)

        @pl.when(jnp.logical_and(i + 2 < NCHUNK, lax.rem(i, 2) == 0))
        def _():
            fetch_idx(i + 2, lax.rem(i + 2, 4), semi0)

        @pl.when(jnp.logical_and(i + 2 < NCHUNK, lax.rem(i, 2) == 1))
        def _():
            fetch_idx(i + 2, lax.rem(i + 2, 4), semi1)

        @pl.when(i + 1 < NCHUNK)
        def _():
            @pl.when(lax.rem(i + 1, 2) == 0)
            def _():
                wait_idx(i + 1, bi1, semi0)

            @pl.when(lax.rem(i + 1, 2) == 1)
            def _():
                wait_idx(i + 1, bi1, semi1)

            @pl.when(i >= 1)
            def _():
                wait_scatter(lax.rem(i + 3, 4), 1 - b)

            fetch_rows(bi1, 1 - b)

        drain_rows(bi, b)

        @plsc.parallel_loop(0, CHUNK, unroll=4)
        def edge_body(e):
            zvec = jnp.zeros((16,), jnp.float32)
            for h in range(N_HEADS):
                kv = kr[b, e, pl.ds(h * 16, 16)]
                qv = qr[b, e, pl.ds(h * 16, 16)]
                p = plsc.cumsum(kv * qv)
                s = jnp.clip(p.at[last].get(mode="promise_in_bounds"),
                             -5.0, 5.0)
                sv = jnp.exp(s)
                vr[b, e, pl.ds(h * 16, 16)] = vr[b, e, pl.ds(h * 16, 16)] * sv
                zvec = jnp.where(lanes == h, sv, zvec)
            zmsg[b, e, pl.ds(0, 16)] = zvec

        start_scatter(bi, b)
        return carry

    lax.fori_loop(0, NCHUNK, chunk_body, 0)
    wait_scatter((NCHUNK - 1) % 4, (NCHUNK - 1) % 2)
    plsc.subcore_barrier()

    pltpu.sync_copy(accwv.at[pl.ds(sid * RPT, RPT)],
                    outwv_hbm.at[cid, pl.ds(sid * RPT, RPT)])
    pltpu.sync_copy(accz.at[pl.ds(sid * RPT, RPT)],
                    outz_hbm.at[cid, pl.ds(sid * RPT, RPT)])


def _combine_body(pwv_ref, pz_ref, b_ref, o_ref):
    wv = pwv_ref[0] + pwv_ref[1]
    z = pz_ref[0, :, :N_HEADS] + pz_ref[1, :, :N_HEADS]
    zb = jnp.dot(z, b_ref[...], preferred_element_type=jnp.float32)
    o_ref[...] = wv / (zb + 1e-6)


def _combine(pwv, pz, bmat):
    return pl.pallas_call(
        _combine_body,
        out_shape=jax.ShapeDtypeStruct((N_NODES, ROW), jnp.float32),
    )(pwv, pz, bmat)


_BMAT = np.zeros((N_HEADS, ROW), dtype=np.float32)
for _h in range(N_HEADS):
    _BMAT[_h, _h * OUT_DIM:(_h + 1) * OUT_DIM] = 1.0
_BMAT.setflags(write=False)


def kernel(x, edge_index, Wq, Wk, Wv):
    q, k, v = _qkv(x, Wq, Wk * (1.0 / np.sqrt(OUT_DIM)), Wv)
    pwv, pz = _sc_attn(k, q, v, edge_index)
    out = _combine(pwv, pz, jnp.asarray(_BMAT))
    return out.reshape(N_NODES, N_HEADS, OUT_DIM)


# speedup vs baseline: 4.5953x; 1.0052x over previous
import functools

import jax
import jax.numpy as jnp
import numpy as np
from jax import lax
from jax.experimental import pallas as pl
from jax.experimental.pallas import tpu as pltpu
from jax.experimental.pallas import tpu_sc as plsc

N_NODES = 10000
N_EDGES = 320000
IN_DIM = 128
OUT_DIM = 16
N_HEADS = 8
ROW = OUT_DIM * N_HEADS
ZROW = ROW + 16

NC = 2
NS = 16
NW = NC * NS
EPW = N_EDGES // NW
CHUNK = 40
NCHUNK = EPW // CHUNK
RPT = N_NODES // NS


def _qkv_body(x_ref, wq_ref, wk_ref, wv_ref, q_ref, k_ref, v_ref):
    xb = x_ref[...]
    q_ref[...] = jnp.dot(xb, wq_ref[...], preferred_element_type=jnp.float32)
    k_ref[...] = jnp.dot(xb, wk_ref[...],
                         preferred_element_type=jnp.float32) * 0.25
    v_ref[...] = jnp.dot(xb, wv_ref[...], preferred_element_type=jnp.float32)


def _qkv(x, Wq, Wk, Wv):
    blk = 2000
    grid = (N_NODES // blk,)
    out = jax.ShapeDtypeStruct((N_NODES, ROW), jnp.float32)
    return pl.pallas_call(
        _qkv_body,
        grid=grid,
        in_specs=[
            pl.BlockSpec((blk, IN_DIM), lambda i: (i, 0)),
            pl.BlockSpec((IN_DIM, ROW), lambda i: (0, 0)),
            pl.BlockSpec((IN_DIM, ROW), lambda i: (0, 0)),
            pl.BlockSpec((IN_DIM, ROW), lambda i: (0, 0)),
        ],
        out_specs=[
            pl.BlockSpec((blk, ROW), lambda i: (i, 0)),
            pl.BlockSpec((blk, ROW), lambda i: (i, 0)),
            pl.BlockSpec((blk, ROW), lambda i: (i, 0)),
        ],
        out_shape=[out, out, out],
    )(x, Wq, Wk, Wv)


_sc_mesh = plsc.VectorSubcoreMesh(core_axis_name="c", subcore_axis_name="s")


@functools.partial(
    pl.kernel,
    mesh=_sc_mesh,
    compiler_params=pltpu.CompilerParams(use_tc_tiling_on_sc=False,
                                         needs_layout_passes=False),
    out_type=(jax.ShapeDtypeStruct((NC, N_NODES, ROW), jnp.float32),
              jax.ShapeDtypeStruct((NC, N_NODES, 16), jnp.float32)),
    scratch_types=[
        pltpu.VMEM((4, CHUNK), jnp.int32),
        pltpu.VMEM((4, CHUNK), jnp.int32),
        pltpu.VMEM((2, CHUNK, ROW), jnp.float32),
        pltpu.VMEM((2, CHUNK, ROW), jnp.float32),
        pltpu.VMEM((2, CHUNK, ROW), jnp.float32),
        pltpu.VMEM((2, CHUNK, 16), jnp.float32),
        pltpu.VMEM_SHARED((N_NODES, ROW), jnp.float32),
        pltpu.VMEM_SHARED((N_NODES, 16), jnp.float32),
        pltpu.SemaphoreType.DMA,
        pltpu.SemaphoreType.DMA,
        pltpu.SemaphoreType.DMA,
        pltpu.SemaphoreType.DMA,
        pltpu.SemaphoreType.DMA,
        pltpu.SemaphoreType.DMA,
        pltpu.SemaphoreType.DMA,
    ],
)
def _sc_attn(k_hbm, q_hbm, v_hbm, edge_hbm, outwv_hbm, outz_hbm,
             srcv, dstv, kr, qr, vr, zmsg, accwv, accz,
             sem1, sem2, sem3, semi0, semi1, semw, semz):
    cid = lax.axis_index("c")
    sid = lax.axis_index("s")
    wid = sid * NC + cid

    zeros16 = jnp.zeros((16,), jnp.float32)

    def zero_row(r, carry):
        for j in range(ROW // 16):
            vr[0, r, pl.ds(j * 16, 16)] = zeros16
        zmsg[0, r, pl.ds(0, 16)] = zeros16
        return carry

    lax.fori_loop(0, CHUNK, zero_row, 0)

    def zero_chunk(j, carry):
        c = sid + j * NS

        @pl.when(c < N_NODES // CHUNK)
        def _():
            pltpu.sync_copy(vr.at[0], accwv.at[pl.ds(c * CHUNK, CHUNK)])
            pltpu.sync_copy(zmsg.at[0], accz.at[pl.ds(c * CHUNK, CHUNK)])

        return carry

    lax.fori_loop(0, (N_NODES // CHUNK + NS - 1) // NS, zero_chunk, 0)
    plsc.subcore_barrier()

    lanes = lax.iota(jnp.int32, 16)
    last = jnp.full((16,), 15, jnp.int32)

    def fetch_idx(i, bi, semi):
        base = wid * EPW + i * CHUNK
        pltpu.async_copy(edge_hbm.at[0, pl.ds(base, CHUNK)], srcv.at[bi], semi)
        pltpu.async_copy(edge_hbm.at[1, pl.ds(base, CHUNK)], dstv.at[bi], semi)

    def wait_idx(i, bi, semi):
        base = wid * EPW + i * CHUNK
        pltpu.make_async_copy(edge_hbm.at[0, pl.ds(base, CHUNK)],
                              srcv.at[bi], semi).wait()
        pltpu.make_async_copy(edge_hbm.at[1, pl.ds(base, CHUNK)],
                              dstv.at[bi], semi).wait()

    def fetch_rows(bi, b):
        pltpu.async_copy(k_hbm.at[srcv.at[bi]], kr.at[b], sem1)
        pltpu.async_copy(q_hbm.at[dstv.at[bi]], qr.at[b], sem2)
        pltpu.async_copy(v_hbm.at[srcv.at[bi]], vr.at[b], sem3)

    def drain_rows(bi, b):
        pltpu.make_async_copy(k_hbm.at[srcv.at[bi]], kr.at[b], sem1).wait()
        pltpu.make_async_copy(q_hbm.at[dstv.at[bi]], qr.at[b], sem2).wait()
        pltpu.make_async_copy(v_hbm.at[srcv.at[bi]], vr.at[b], sem3).wait()

    def start_scatter(bi, b):
        pltpu.async_copy(vr.at[b], accwv.at[dstv.at[bi]], semw, add=True)
        pltpu.async_copy(zmsg.at[b], accz.at[dstv.at[bi]], semz, add=True)

    def wait_scatter(bi, b):
        pltpu.make_async_copy(vr.at[b], accwv.at[dstv.at[bi]], semw).wait()
        pltpu.make_async_copy(zmsg.at[b], accz.at[dstv.at[bi]], semz).wait()

    fetch_idx(0, 0, semi0)
    fetch_idx(1, 1, semi1)
    wait_idx(0, 0, semi0)
    fetch_rows(0, 0)

    def chunk_body(i, carry):
        b = lax.rem(i, 2)
        bi = lax.rem(i, 4)
        bi1 = lax.rem(i + 1, 4)

        @pl.when(jnp.logical_and(i + 2 < NCHUNK, lax.rem(i, 2) == 0))
        def _():
            fetch_idx(i + 2, lax.rem(i + 2, 4), semi0)

        @pl.when(jnp.logical_and(i + 2 < NCHUNK, lax.rem(i, 2) == 1))
        def _():
            fetch_idx(i + 2, lax.rem(i + 2, 4), semi1)

        @pl.when(i + 1 < NCHUNK)
        def _():
            @pl.when(lax.rem(i + 1, 2) == 0)
            def _():
                wait_idx(i + 1, bi1, semi0)

            @pl.when(lax.rem(i + 1, 2) == 1)
            def _():
                wait_idx(i + 1, bi1, semi1)

            @pl.when(i >= 1)
            def _():
                wait_scatter(lax.rem(i + 3, 4), 1 - b)

            fetch_rows(bi1, 1 - b)

        drain_rows(bi, b)

        @plsc.parallel_loop(0, CHUNK, unroll=4)
        def edge_body(e):
            zvec = jnp.zeros((16,), jnp.float32)
            for h in range(N_HEADS):
                kv = kr[b, e, pl.ds(h * 16, 16)]
                qv = qr[b, e, pl.ds(h * 16, 16)]
                p = plsc.cumsum(kv * qv)
                s = jnp.clip(p.at[last].get(mode="promise_in_bounds"),
                             -5.0, 5.0)
                sv = jnp.exp(s)
                vr[b, e, pl.ds(h * 16, 16)] = vr[b, e, pl.ds(h * 16, 16)] * sv
                zvec = jnp.where(lanes == h, sv, zvec)
            zmsg[b, e, pl.ds(0, 16)] = zvec

        start_scatter(bi, b)
        return carry

    lax.fori_loop(0, NCHUNK, chunk_body, 0)
    wait_scatter((NCHUNK - 1) % 4, (NCHUNK - 1) % 2)
    plsc.subcore_barrier()

    pltpu.sync_copy(accwv.at[pl.ds(sid * RPT, RPT)],
                    outwv_hbm.at[cid, pl.ds(sid * RPT, RPT)])
    pltpu.sync_copy(accz.at[pl.ds(sid * RPT, RPT)],
                    outz_hbm.at[cid, pl.ds(sid * RPT, RPT)])


def _combine_body(pwv_ref, pz_ref, b_ref, o_ref):
    wv = pwv_ref[0] + pwv_ref[1]
    z = pz_ref[0, :, :N_HEADS] + pz_ref[1, :, :N_HEADS]
    zb = jnp.dot(z, b_ref[...], preferred_element_type=jnp.float32)
    o_ref[...] = wv / (zb + 1e-6)


def _combine(pwv, pz, bmat):
    blk = 2000
    return pl.pallas_call(
        _combine_body,
        grid=(N_NODES // blk,),
        in_specs=[
            pl.BlockSpec((NC, blk, ROW), lambda i: (0, i, 0)),
            pl.BlockSpec((NC, blk, 16), lambda i: (0, i, 0)),
            pl.BlockSpec((N_HEADS, ROW), lambda i: (0, 0)),
        ],
        out_specs=pl.BlockSpec((blk, ROW), lambda i: (i, 0)),
        out_shape=jax.ShapeDtypeStruct((N_NODES, ROW), jnp.float32),
    )(pwv, pz, bmat)


_BMAT = np.zeros((N_HEADS, ROW), dtype=np.float32)
for _h in range(N_HEADS):
    _BMAT[_h, _h * OUT_DIM:(_h + 1) * OUT_DIM] = 1.0
_BMAT.setflags(write=False)


def kernel(x, edge_index, Wq, Wk, Wv):
    q, k, v = _qkv(x, Wq, Wk, Wv)
    pwv, pz = _sc_attn(k, q, v, edge_index)
    out = _combine(pwv, pz, jnp.asarray(_BMAT))
    return out.reshape(N_NODES, N_HEADS, OUT_DIM)


# speedup vs baseline: 5.0806x; 1.1056x over previous
import functools

import jax
import jax.numpy as jnp
import numpy as np
from jax import lax
from jax.experimental import pallas as pl
from jax.experimental.pallas import tpu as pltpu
from jax.experimental.pallas import tpu_sc as plsc

N_NODES = 10000
N_EDGES = 320000
IN_DIM = 128
OUT_DIM = 16
N_HEADS = 8
ROW = OUT_DIM * N_HEADS
ZROW = ROW + 16

NC = 2
NS = 16
NW = NC * NS
EPW = N_EDGES // NW
CHUNK = 40
NCHUNK = EPW // CHUNK
RPT = N_NODES // NS


def _qkv_body(x_ref, wq_ref, wk_ref, wv_ref, q_ref, k_ref, v_ref):
    xb = x_ref[...]
    q_ref[...] = jnp.dot(xb, wq_ref[...], preferred_element_type=jnp.float32)
    k_ref[...] = jnp.dot(xb, wk_ref[...],
                         preferred_element_type=jnp.float32) * 0.25
    v_ref[...] = jnp.dot(xb, wv_ref[...], preferred_element_type=jnp.float32)


def _qkv(x, Wq, Wk, Wv):
    blk = 2000
    grid = (N_NODES // blk,)
    out = jax.ShapeDtypeStruct((N_NODES, ROW), jnp.float32)
    return pl.pallas_call(
        _qkv_body,
        grid=grid,
        in_specs=[
            pl.BlockSpec((blk, IN_DIM), lambda i: (i, 0)),
            pl.BlockSpec((IN_DIM, ROW), lambda i: (0, 0)),
            pl.BlockSpec((IN_DIM, ROW), lambda i: (0, 0)),
            pl.BlockSpec((IN_DIM, ROW), lambda i: (0, 0)),
        ],
        out_specs=[
            pl.BlockSpec((blk, ROW), lambda i: (i, 0)),
            pl.BlockSpec((blk, ROW), lambda i: (i, 0)),
            pl.BlockSpec((blk, ROW), lambda i: (i, 0)),
        ],
        out_shape=[out, out, out],
    )(x, Wq, Wk, Wv)


_sc_mesh = plsc.VectorSubcoreMesh(core_axis_name="c", subcore_axis_name="s")


@functools.partial(
    pl.kernel,
    mesh=_sc_mesh,
    compiler_params=pltpu.CompilerParams(use_tc_tiling_on_sc=False,
                                         needs_layout_passes=False),
    out_type=(jax.ShapeDtypeStruct((NC, N_NODES, ROW), jnp.float32),
              jax.ShapeDtypeStruct((NC, N_NODES, 16), jnp.float32)),
    scratch_types=[
        pltpu.VMEM((4, CHUNK), jnp.int32),
        pltpu.VMEM((4, CHUNK), jnp.int32),
        pltpu.VMEM((2, CHUNK, ROW), jnp.float32),
        pltpu.VMEM((2, CHUNK, ROW), jnp.float32),
        pltpu.VMEM((2, CHUNK, ROW), jnp.float32),
        pltpu.VMEM((2, CHUNK, 16), jnp.float32),
        pltpu.VMEM_SHARED((N_NODES, ROW), jnp.float32),
        pltpu.VMEM_SHARED((N_NODES, 16), jnp.float32),
        pltpu.SemaphoreType.DMA,
        pltpu.SemaphoreType.DMA,
        pltpu.SemaphoreType.DMA,
        pltpu.SemaphoreType.DMA,
        pltpu.SemaphoreType.DMA,
        pltpu.SemaphoreType.DMA,
        pltpu.SemaphoreType.DMA,
    ],
)
def _sc_attn(k_hbm, q_hbm, v_hbm, edge_hbm, outwv_hbm, outz_hbm,
             srcv, dstv, kr, qr, vr, zmsg, accwv, accz,
             sem1, sem2, sem3, semi0, semi1, semw, semz):
    cid = lax.axis_index("c")
    sid = lax.axis_index("s")
    wid = sid * NC + cid

    zeros16 = jnp.zeros((16,), jnp.float32)

    def zero_row(r, carry):
        for j in range(ROW // 16):
            vr[0, r, pl.ds(j * 16, 16)] = zeros16
        zmsg[0, r, pl.ds(0, 16)] = zeros16
        return carry

    lax.fori_loop(0, CHUNK, zero_row, 0)

    def zero_chunk(j, carry):
        c = sid + j * NS

        @pl.when(c < N_NODES // CHUNK)
        def _():
            pltpu.sync_copy(vr.at[0], accwv.at[pl.ds(c * CHUNK, CHUNK)])
            pltpu.sync_copy(zmsg.at[0], accz.at[pl.ds(c * CHUNK, CHUNK)])

        return carry

    lax.fori_loop(0, (N_NODES // CHUNK + NS - 1) // NS, zero_chunk, 0)
    plsc.subcore_barrier()

    lanes = lax.iota(jnp.int32, 16)
    hsel = [jnp.full((16,), h, jnp.int32) for h in range(N_HEADS)]

    def fetch_idx(i, bi, semi):
        base = wid * EPW + i * CHUNK
        pltpu.async_copy(edge_hbm.at[0, pl.ds(base, CHUNK)], srcv.at[bi], semi)
        pltpu.async_copy(edge_hbm.at[1, pl.ds(base, CHUNK)], dstv.at[bi], semi)

    def wait_idx(i, bi, semi):
        base = wid * EPW + i * CHUNK
        pltpu.make_async_copy(edge_hbm.at[0, pl.ds(base, CHUNK)],
                              srcv.at[bi], semi).wait()
        pltpu.make_async_copy(edge_hbm.at[1, pl.ds(base, CHUNK)],
                              dstv.at[bi], semi).wait()

    def fetch_rows(bi, b):
        pltpu.async_copy(k_hbm.at[srcv.at[bi]], kr.at[b], sem1)
        pltpu.async_copy(q_hbm.at[dstv.at[bi]], qr.at[b], sem2)
        pltpu.async_copy(v_hbm.at[srcv.at[bi]], vr.at[b], sem3)

    def drain_rows(bi, b):
        pltpu.make_async_copy(k_hbm.at[srcv.at[bi]], kr.at[b], sem1).wait()
        pltpu.make_async_copy(q_hbm.at[dstv.at[bi]], qr.at[b], sem2).wait()
        pltpu.make_async_copy(v_hbm.at[srcv.at[bi]], vr.at[b], sem3).wait()

    def start_scatter(bi, b):
        pltpu.async_copy(vr.at[b], accwv.at[dstv.at[bi]], semw, add=True)
        pltpu.async_copy(zmsg.at[b], accz.at[dstv.at[bi]], semz, add=True)

    def wait_scatter(bi, b):
        pltpu.make_async_copy(vr.at[b], accwv.at[dstv.at[bi]], semw).wait()
        pltpu.make_async_copy(zmsg.at[b], accz.at[dstv.at[bi]], semz).wait()

    fetch_idx(0, 0, semi0)
    fetch_idx(1, 1, semi1)
    wait_idx(0, 0, semi0)
    fetch_rows(0, 0)

    def chunk_body(i, carry):
        b = lax.rem(i, 2)
        bi = lax.rem(i, 4)
        bi1 = lax.rem(i + 1, 4)

        @pl.when(jnp.logical_and(i + 2 < NCHUNK, lax.rem(i, 2) == 0))
        def _():
            fetch_idx(i + 2, lax.rem(i + 2, 4), semi0)

        @pl.when(jnp.logical_and(i + 2 < NCHUNK, lax.rem(i, 2) == 1))
        def _():
            fetch_idx(i + 2, lax.rem(i + 2, 4), semi1)

        @pl.when(i + 1 < NCHUNK)
        def _():
            @pl.when(lax.rem(i + 1, 2) == 0)
            def _():
                wait_idx(i + 1, bi1, semi0)

            @pl.when(lax.rem(i + 1, 2) == 1)
            def _():
                wait_idx(i + 1, bi1, semi1)

            @pl.when(i >= 1)
            def _():
                wait_scatter(lax.rem(i + 3, 4), 1 - b)

            fetch_rows(bi1, 1 - b)

        drain_rows(bi, b)

        @plsc.parallel_loop(0, CHUNK, unroll=4)
        def edge_body(e):
            zvec = jnp.zeros((16,), jnp.float32)
            for h in range(N_HEADS):
                kv = kr[b, e, pl.ds(h * 16, 16)]
                qv = qr[b, e, pl.ds(h * 16, 16)]
                s = jnp.clip(jnp.sum(kv * qv), -5.0, 5.0)
                zvec = jnp.where(lanes == h, s, zvec)
            ez = jnp.exp(zvec)
            zmsg[b, e, pl.ds(0, 16)] = ez
            for h in range(N_HEADS):
                sv = ez.at[hsel[h]].get(mode="promise_in_bounds")
                vr[b, e, pl.ds(h * 16, 16)] = vr[b, e, pl.ds(h * 16, 16)] * sv

        start_scatter(bi, b)
        return carry

    lax.fori_loop(0, NCHUNK, chunk_body, 0)
    wait_scatter((NCHUNK - 1) % 4, (NCHUNK - 1) % 2)
    plsc.subcore_barrier()

    pltpu.sync_copy(accwv.at[pl.ds(sid * RPT, RPT)],
                    outwv_hbm.at[cid, pl.ds(sid * RPT, RPT)])
    pltpu.sync_copy(accz.at[pl.ds(sid * RPT, RPT)],
                    outz_hbm.at[cid, pl.ds(sid * RPT, RPT)])


def _combine_body(pwv_ref, pz_ref, b_ref, o_ref):
    wv = pwv_ref[0] + pwv_ref[1]
    z = pz_ref[0, :, :N_HEADS] + pz_ref[1, :, :N_HEADS]
    zb = jnp.dot(z, b_ref[...], preferred_element_type=jnp.float32)
    o_ref[...] = wv / (zb + 1e-6)


def _combine(pwv, pz, bmat):
    blk = 2000
    return pl.pallas_call(
        _combine_body,
        grid=(N_NODES // blk,),
        in_specs=[
            pl.BlockSpec((NC, blk, ROW), lambda i: (0, i, 0)),
            pl.BlockSpec((NC, blk, 16), lambda i: (0, i, 0)),
            pl.BlockSpec((N_HEADS, ROW), lambda i: (0, 0)),
        ],
        out_specs=pl.BlockSpec((blk, ROW), lambda i: (i, 0)),
        out_shape=jax.ShapeDtypeStruct((N_NODES, ROW), jnp.float32),
    )(pwv, pz, bmat)


_BMAT = np.zeros((N_HEADS, ROW), dtype=np.float32)
for _h in range(N_HEADS):
    _BMAT[_h, _h * OUT_DIM:(_h + 1) * OUT_DIM] = 1.0
_BMAT.setflags(write=False)


def kernel(x, edge_index, Wq, Wk, Wv):
    q, k, v = _qkv(x, Wq, Wk, Wv)
    pwv, pz = _sc_attn(k, q, v, edge_index)
    out = _combine(pwv, pz, jnp.asarray(_BMAT))
    return out.reshape(N_NODES, N_HEADS, OUT_DIM)


# speedup vs baseline: 5.4604x; 1.0748x over previous
import functools

import jax
import jax.numpy as jnp
import numpy as np
from jax import lax
from jax.experimental import pallas as pl
from jax.experimental.pallas import tpu as pltpu
from jax.experimental.pallas import tpu_sc as plsc

N_NODES = 10000
N_EDGES = 320000
IN_DIM = 128
OUT_DIM = 16
N_HEADS = 8
ROW = OUT_DIM * N_HEADS
ZROW = ROW + 16

NC = 2
NS = 16
NW = NC * NS
EPW = N_EDGES // NW
CHUNK = 40
NCHUNK = EPW // CHUNK
RPT = N_NODES // NS


def _qkv_body(x_ref, wq_ref, wk_ref, wv_ref, q_ref, k_ref, v_ref):
    xb = x_ref[...]
    q_ref[...] = jnp.dot(
        xb, wq_ref[...], preferred_element_type=jnp.float32
    ).astype(jnp.bfloat16)
    k_ref[...] = (jnp.dot(xb, wk_ref[...],
                          preferred_element_type=jnp.float32) * 0.25
                  ).astype(jnp.bfloat16)
    v_ref[...] = jnp.dot(xb, wv_ref[...], preferred_element_type=jnp.float32)


def _qkv(x, Wq, Wk, Wv):
    blk = 2000
    grid = (N_NODES // blk,)
    outh = jax.ShapeDtypeStruct((N_NODES, ROW), jnp.bfloat16)
    outf = jax.ShapeDtypeStruct((N_NODES, ROW), jnp.float32)
    return pl.pallas_call(
        _qkv_body,
        grid=grid,
        in_specs=[
            pl.BlockSpec((blk, IN_DIM), lambda i: (i, 0)),
            pl.BlockSpec((IN_DIM, ROW), lambda i: (0, 0)),
            pl.BlockSpec((IN_DIM, ROW), lambda i: (0, 0)),
            pl.BlockSpec((IN_DIM, ROW), lambda i: (0, 0)),
        ],
        out_specs=[
            pl.BlockSpec((blk, ROW), lambda i: (i, 0)),
            pl.BlockSpec((blk, ROW), lambda i: (i, 0)),
            pl.BlockSpec((blk, ROW), lambda i: (i, 0)),
        ],
        out_shape=[outh, outh, outf],
    )(x, Wq, Wk, Wv)


_sc_mesh = plsc.VectorSubcoreMesh(core_axis_name="c", subcore_axis_name="s")


@functools.partial(
    pl.kernel,
    mesh=_sc_mesh,
    compiler_params=pltpu.CompilerParams(use_tc_tiling_on_sc=False,
                                         needs_layout_passes=False),
    out_type=(jax.ShapeDtypeStruct((NC, N_NODES, ROW), jnp.float32),
              jax.ShapeDtypeStruct((NC, N_NODES, 16), jnp.float32)),
    scratch_types=[
        pltpu.VMEM((4, CHUNK), jnp.int32),
        pltpu.VMEM((4, CHUNK), jnp.int32),
        pltpu.VMEM((2, CHUNK, ROW), jnp.bfloat16),
        pltpu.VMEM((2, CHUNK, ROW), jnp.bfloat16),
        pltpu.VMEM((2, CHUNK, ROW), jnp.float32),
        pltpu.VMEM((2, CHUNK, 16), jnp.float32),
        pltpu.VMEM_SHARED((N_NODES, ROW), jnp.float32),
        pltpu.VMEM_SHARED((N_NODES, 16), jnp.float32),
        pltpu.SemaphoreType.DMA,
        pltpu.SemaphoreType.DMA,
        pltpu.SemaphoreType.DMA,
        pltpu.SemaphoreType.DMA,
        pltpu.SemaphoreType.DMA,
        pltpu.SemaphoreType.DMA,
        pltpu.SemaphoreType.DMA,
    ],
)
def _sc_attn(k_hbm, q_hbm, v_hbm, edge_hbm, outwv_hbm, outz_hbm,
             srcv, dstv, kr, qr, vr, zmsg, accwv, accz,
             sem1, sem2, sem3, semi0, semi1, semw, semz):
    cid = lax.axis_index("c")
    sid = lax.axis_index("s")
    wid = sid * NC + cid

    zeros16 = jnp.zeros((16,), jnp.float32)

    def zero_row(r, carry):
        for j in range(ROW // 16):
            vr[0, r, pl.ds(j * 16, 16)] = zeros16
        zmsg[0, r, pl.ds(0, 16)] = zeros16
        return carry

    lax.fori_loop(0, CHUNK, zero_row, 0)

    def zero_chunk(j, carry):
        c = sid + j * NS

        @pl.when(c < N_NODES // CHUNK)
        def _():
            pltpu.sync_copy(vr.at[0], accwv.at[pl.ds(c * CHUNK, CHUNK)])
            pltpu.sync_copy(zmsg.at[0], accz.at[pl.ds(c * CHUNK, CHUNK)])

        return carry

    lax.fori_loop(0, (N_NODES // CHUNK + NS - 1) // NS, zero_chunk, 0)
    plsc.subcore_barrier()

    lanes = lax.iota(jnp.int32, 16)
    hsel = [jnp.full((16,), h, jnp.int32) for h in range(N_HEADS)]

    def fetch_idx(i, bi, semi):
        base = wid * EPW + i * CHUNK
        pltpu.async_copy(edge_hbm.at[0, pl.ds(base, CHUNK)], srcv.at[bi], semi)
        pltpu.async_copy(edge_hbm.at[1, pl.ds(base, CHUNK)], dstv.at[bi], semi)

    def wait_idx(i, bi, semi):
        base = wid * EPW + i * CHUNK
        pltpu.make_async_copy(edge_hbm.at[0, pl.ds(base, CHUNK)],
                              srcv.at[bi], semi).wait()
        pltpu.make_async_copy(edge_hbm.at[1, pl.ds(base, CHUNK)],
                              dstv.at[bi], semi).wait()

    def fetch_rows(bi, b):
        pltpu.async_copy(k_hbm.at[srcv.at[bi]], kr.at[b], sem1)
        pltpu.async_copy(q_hbm.at[dstv.at[bi]], qr.at[b], sem2)
        pltpu.async_copy(v_hbm.at[srcv.at[bi]], vr.at[b], sem3)

    def drain_rows(bi, b):
        pltpu.make_async_copy(k_hbm.at[srcv.at[bi]], kr.at[b], sem1).wait()
        pltpu.make_async_copy(q_hbm.at[dstv.at[bi]], qr.at[b], sem2).wait()
        pltpu.make_async_copy(v_hbm.at[srcv.at[bi]], vr.at[b], sem3).wait()

    def start_scatter(bi, b):
        pltpu.async_copy(vr.at[b], accwv.at[dstv.at[bi]], semw, add=True)
        pltpu.async_copy(zmsg.at[b], accz.at[dstv.at[bi]], semz, add=True)

    def wait_scatter(bi, b):
        pltpu.make_async_copy(vr.at[b], accwv.at[dstv.at[bi]], semw).wait()
        pltpu.make_async_copy(zmsg.at[b], accz.at[dstv.at[bi]], semz).wait()

    fetch_idx(0, 0, semi0)
    fetch_idx(1, 1, semi1)
    wait_idx(0, 0, semi0)
    fetch_rows(0, 0)

    def chunk_body(i, carry):
        b = lax.rem(i, 2)
        bi = lax.rem(i, 4)
        bi1 = lax.rem(i + 1, 4)

        @pl.when(jnp.logical_and(i + 2 < NCHUNK, lax.rem(i, 2) == 0))
        def _():
            fetch_idx(i + 2, lax.rem(i + 2, 4), semi0)

        @pl.when(jnp.logical_and(i + 2 < NCHUNK, lax.rem(i, 2) == 1))
        def _():
            fetch_idx(i + 2, lax.rem(i + 2, 4), semi1)

        @pl.when(i + 1 < NCHUNK)
        def _():
            @pl.when(lax.rem(i + 1, 2) == 0)
            def _():
                wait_idx(i + 1, bi1, semi0)

            @pl.when(lax.rem(i + 1, 2) == 1)
            def _():
                wait_idx(i + 1, bi1, semi1)

            @pl.when(i >= 1)
            def _():
                wait_scatter(lax.rem(i + 3, 4), 1 - b)

            fetch_rows(bi1, 1 - b)

        drain_rows(bi, b)

        @plsc.parallel_loop(0, CHUNK, unroll=4)
        def edge_body(e):
            zvec = jnp.zeros((16,), jnp.float32)
            for g in range(N_HEADS // 2):
                kv2 = kr[b, e, pl.ds(32 * g, 32)]
                qv2 = qr[b, e, pl.ds(32 * g, 32)]
                ka, kb = plsc.unpack(kv2, format=plsc.PackFormat.INTERLEAVED)
                qa, qb = plsc.unpack(qv2, format=plsc.PackFormat.INTERLEAVED)
                for h, kv, qv in ((2 * g, ka, qa), (2 * g + 1, kb, qb)):
                    s = jnp.clip(jnp.sum(kv * qv), -5.0, 5.0)
                    zvec = jnp.where(lanes == h, s, zvec)
            ez = jnp.exp(zvec)
            zmsg[b, e, pl.ds(0, 16)] = ez
            for h in range(N_HEADS):
                sv = ez.at[hsel[h]].get(mode="promise_in_bounds")
                vr[b, e, pl.ds(h * 16, 16)] = vr[b, e, pl.ds(h * 16, 16)] * sv

        start_scatter(bi, b)
        return carry

    lax.fori_loop(0, NCHUNK, chunk_body, 0)
    wait_scatter((NCHUNK - 1) % 4, (NCHUNK - 1) % 2)
    plsc.subcore_barrier()

    pltpu.sync_copy(accwv.at[pl.ds(sid * RPT, RPT)],
                    outwv_hbm.at[cid, pl.ds(sid * RPT, RPT)])
    pltpu.sync_copy(accz.at[pl.ds(sid * RPT, RPT)],
                    outz_hbm.at[cid, pl.ds(sid * RPT, RPT)])


def _combine_body(pwv_ref, pz_ref, b_ref, o_ref):
    wv = pwv_ref[0] + pwv_ref[1]
    z = pz_ref[0, :, :N_HEADS] + pz_ref[1, :, :N_HEADS]
    zb = jnp.dot(z, b_ref[...], preferred_element_type=jnp.float32)
    o_ref[...] = wv / (zb + 1e-6)


def _combine(pwv, pz, bmat):
    blk = 2000
    return pl.pallas_call(
        _combine_body,
        grid=(N_NODES // blk,),
        in_specs=[
            pl.BlockSpec((NC, blk, ROW), lambda i: (0, i, 0)),
            pl.BlockSpec((NC, blk, 16), lambda i: (0, i, 0)),
            pl.BlockSpec((N_HEADS, ROW), lambda i: (0, 0)),
        ],
        out_specs=pl.BlockSpec((blk, ROW), lambda i: (i, 0)),
        out_shape=jax.ShapeDtypeStruct((N_NODES, ROW), jnp.float32),
    )(pwv, pz, bmat)


_BMAT = np.zeros((N_HEADS, ROW), dtype=np.float32)
for _h in range(N_HEADS):
    _BMAT[_h, _h * OUT_DIM:(_h + 1) * OUT_DIM] = 1.0
_BMAT.setflags(write=False)

_PERM = np.zeros((ROW,), dtype=np.int32)
for _g in range(N_HEADS // 2):
    for _d in range(OUT_DIM):
        _PERM[32 * _g + 2 * _d] = 32 * _g + _d
        _PERM[32 * _g + 2 * _d + 1] = 32 * _g + OUT_DIM + _d
_PERM.setflags(write=False)


def kernel(x, edge_index, Wq, Wk, Wv):
    perm = jnp.asarray(_PERM)
    q, k, v = _qkv(x, Wq[:, perm], Wk[:, perm], Wv)
    pwv, pz = _sc_attn(k, q, v, edge_index)
    out = _combine(pwv, pz, jnp.asarray(_BMAT))
    return out.reshape(N_NODES, N_HEADS, OUT_DIM)


# speedup vs baseline: 5.4777x; 1.0032x over previous
import functools

import jax
import jax.numpy as jnp
import numpy as np
from jax import lax
from jax.experimental import pallas as pl
from jax.experimental.pallas import tpu as pltpu
from jax.experimental.pallas import tpu_sc as plsc

N_NODES = 10000
N_EDGES = 320000
IN_DIM = 128
OUT_DIM = 16
N_HEADS = 8
ROW = OUT_DIM * N_HEADS
ZROW = ROW + 16

NC = 2
NS = 16
NW = NC * NS
EPW = N_EDGES // NW
CHUNK = 40
NCHUNK = EPW // CHUNK
RPT = N_NODES // NS


def _qkv_body(x_ref, wq_ref, wk_ref, wv_ref, q_ref, k_ref, v_ref):
    xb = x_ref[...]
    q_ref[...] = jnp.dot(
        xb, wq_ref[...], preferred_element_type=jnp.float32
    ).astype(jnp.bfloat16)
    k_ref[...] = (jnp.dot(xb, wk_ref[...],
                          preferred_element_type=jnp.float32) * 0.25
                  ).astype(jnp.bfloat16)
    v_ref[...] = jnp.dot(xb, wv_ref[...], preferred_element_type=jnp.float32)


def _qkv(x, Wq, Wk, Wv):
    blk = 2000
    grid = (N_NODES // blk,)
    outh = jax.ShapeDtypeStruct((N_NODES, ROW), jnp.bfloat16)
    outf = jax.ShapeDtypeStruct((N_NODES, ROW), jnp.float32)
    return pl.pallas_call(
        _qkv_body,
        grid=grid,
        in_specs=[
            pl.BlockSpec((blk, IN_DIM), lambda i: (i, 0)),
            pl.BlockSpec((IN_DIM, ROW), lambda i: (0, 0)),
            pl.BlockSpec((IN_DIM, ROW), lambda i: (0, 0)),
            pl.BlockSpec((IN_DIM, ROW), lambda i: (0, 0)),
        ],
        out_specs=[
            pl.BlockSpec((blk, ROW), lambda i: (i, 0)),
            pl.BlockSpec((blk, ROW), lambda i: (i, 0)),
            pl.BlockSpec((blk, ROW), lambda i: (i, 0)),
        ],
        out_shape=[outh, outh, outf],
    )(x, Wq, Wk, Wv)


_sc_mesh = plsc.VectorSubcoreMesh(core_axis_name="c", subcore_axis_name="s")


@functools.partial(
    pl.kernel,
    mesh=_sc_mesh,
    compiler_params=pltpu.CompilerParams(use_tc_tiling_on_sc=False,
                                         needs_layout_passes=False),
    out_type=(jax.ShapeDtypeStruct((NC, N_NODES, ROW), jnp.float32),
              jax.ShapeDtypeStruct((NC, N_NODES, 16), jnp.float32)),
    scratch_types=[
        pltpu.VMEM((4, CHUNK), jnp.int32),
        pltpu.VMEM((4, CHUNK), jnp.int32),
        pltpu.VMEM((2, CHUNK, ROW), jnp.bfloat16),
        pltpu.VMEM((2, CHUNK, ROW), jnp.bfloat16),
        pltpu.VMEM((2, CHUNK, ROW), jnp.float32),
        pltpu.VMEM((2, CHUNK, 16), jnp.float32),
        pltpu.VMEM_SHARED((N_NODES, ROW), jnp.float32),
        pltpu.VMEM_SHARED((N_NODES, 16), jnp.float32),
        pltpu.SemaphoreType.DMA,
        pltpu.SemaphoreType.DMA,
        pltpu.SemaphoreType.DMA,
        pltpu.SemaphoreType.DMA,
        pltpu.SemaphoreType.DMA,
        pltpu.SemaphoreType.DMA,
        pltpu.SemaphoreType.DMA,
    ],
)
def _sc_attn(k_hbm, q_hbm, v_hbm, edge_hbm, outwv_hbm, outz_hbm,
             srcv, dstv, kr, qr, vr, zmsg, accwv, accz,
             sem1, sem2, sem3, semi0, semi1, semw, semz):
    cid = lax.axis_index("c")
    sid = lax.axis_index("s")
    wid = sid * NC + cid

    zeros16 = jnp.zeros((16,), jnp.float32)

    def zero_row(r, carry):
        for j in range(ROW // 16):
            vr[0, r, pl.ds(j * 16, 16)] = zeros16
        zmsg[0, r, pl.ds(0, 16)] = zeros16
        return carry

    lax.fori_loop(0, CHUNK, zero_row, 0)

    def zero_chunk(j, carry):
        c = sid + j * NS

        @pl.when(c < N_NODES // CHUNK)
        def _():
            pltpu.sync_copy(vr.at[0], accwv.at[pl.ds(c * CHUNK, CHUNK)])
            pltpu.sync_copy(zmsg.at[0], accz.at[pl.ds(c * CHUNK, CHUNK)])

        return carry

    lax.fori_loop(0, (N_NODES // CHUNK + NS - 1) // NS, zero_chunk, 0)
    plsc.subcore_barrier()

    lanes = lax.iota(jnp.int32, 16)
    hmask = [lanes == h for h in range(N_HEADS)]
    hsel = [jnp.full((16,), h, jnp.int32) for h in range(N_HEADS)]

    def fetch_idx(i, bi, semi):
        base = wid * EPW + i * CHUNK
        pltpu.async_copy(edge_hbm.at[0, pl.ds(base, CHUNK)], srcv.at[bi], semi)
        pltpu.async_copy(edge_hbm.at[1, pl.ds(base, CHUNK)], dstv.at[bi], semi)

    def wait_idx(i, bi, semi):
        base = wid * EPW + i * CHUNK
        pltpu.make_async_copy(edge_hbm.at[0, pl.ds(base, CHUNK)],
                              srcv.at[bi], semi).wait()
        pltpu.make_async_copy(edge_hbm.at[1, pl.ds(base, CHUNK)],
                              dstv.at[bi], semi).wait()

    def fetch_rows(bi, b):
        pltpu.async_copy(k_hbm.at[srcv.at[bi]], kr.at[b], sem1)
        pltpu.async_copy(q_hbm.at[dstv.at[bi]], qr.at[b], sem2)
        pltpu.async_copy(v_hbm.at[srcv.at[bi]], vr.at[b], sem3)

    def drain_rows(bi, b):
        pltpu.make_async_copy(k_hbm.at[srcv.at[bi]], kr.at[b], sem1).wait()
        pltpu.make_async_copy(q_hbm.at[dstv.at[bi]], qr.at[b], sem2).wait()
        pltpu.make_async_copy(v_hbm.at[srcv.at[bi]], vr.at[b], sem3).wait()

    def start_scatter(bi, b):
        pltpu.async_copy(vr.at[b], accwv.at[dstv.at[bi]], semw, add=True)
        pltpu.async_copy(zmsg.at[b], accz.at[dstv.at[bi]], semz, add=True)

    def wait_scatter(bi, b):
        pltpu.make_async_copy(vr.at[b], accwv.at[dstv.at[bi]], semw).wait()
        pltpu.make_async_copy(zmsg.at[b], accz.at[dstv.at[bi]], semz).wait()

    def compute(b, e):
        zvec = jnp.zeros((16,), jnp.float32)
        for g in range(N_HEADS // 2):
            kv2 = kr[b, e, pl.ds(32 * g, 32)]
            qv2 = qr[b, e, pl.ds(32 * g, 32)]
            ka, kb = plsc.unpack(kv2, format=plsc.PackFormat.INTERLEAVED)
            qa, qb = plsc.unpack(qv2, format=plsc.PackFormat.INTERLEAVED)
            for h, kv, qv in ((2 * g, ka, qa), (2 * g + 1, kb, qb)):
                s = jnp.clip(jnp.sum(kv * qv), -5.0, 5.0)
                zvec = jnp.where(hmask[h], s, zvec)
        ez = jnp.exp(zvec)
        zmsg[b, e, pl.ds(0, 16)] = ez
        for h in range(N_HEADS):
            sv = ez.at[hsel[h]].get(mode="promise_in_bounds")
            vr[b, e, pl.ds(h * 16, 16)] = vr[b, e, pl.ds(h * 16, 16)] * sv

    def half_body(i, b, semi_here, semi_next):
        @pl.when(i + 2 < NCHUNK)
        def _():
            fetch_idx(i + 2, lax.rem(i + 2, 4), semi_here)

        @pl.when(i + 1 < NCHUNK)
        def _():
            wait_idx(i + 1, lax.rem(i + 1, 4), semi_next)

            @pl.when(i >= 1)
            def _():
                wait_scatter(lax.rem(i + 3, 4), 1 - b)

            fetch_rows(lax.rem(i + 1, 4), 1 - b)

        drain_rows(lax.rem(i, 4), b)

        @plsc.parallel_loop(0, CHUNK, unroll=4)
        def edge_body(e):
            compute(b, e)

        start_scatter(lax.rem(i, 4), b)

    fetch_idx(0, 0, semi0)
    fetch_idx(1, 1, semi1)
    wait_idx(0, 0, semi0)
    fetch_rows(0, 0)

    def chunk_pair(j, carry):
        i0 = 2 * j
        half_body(i0, 0, semi0, semi1)
        half_body(i0 + 1, 1, semi1, semi0)
        return carry

    lax.fori_loop(0, NCHUNK // 2, chunk_pair, 0)
    wait_scatter((NCHUNK - 1) % 4, (NCHUNK - 1) % 2)
    plsc.subcore_barrier()

    pltpu.sync_copy(accwv.at[pl.ds(sid * RPT, RPT)],
                    outwv_hbm.at[cid, pl.ds(sid * RPT, RPT)])
    pltpu.sync_copy(accz.at[pl.ds(sid * RPT, RPT)],
                    outz_hbm.at[cid, pl.ds(sid * RPT, RPT)])


def _combine_body(pwv_ref, pz_ref, b_ref, o_ref):
    wv = pwv_ref[0] + pwv_ref[1]
    z = pz_ref[0, :, :N_HEADS] + pz_ref[1, :, :N_HEADS]
    zb = jnp.dot(z, b_ref[...], preferred_element_type=jnp.float32)
    o_ref[...] = wv / (zb + 1e-6)


def _combine(pwv, pz, bmat):
    blk = 2000
    return pl.pallas_call(
        _combine_body,
        grid=(N_NODES // blk,),
        in_specs=[
            pl.BlockSpec((NC, blk, ROW), lambda i: (0, i, 0)),
            pl.BlockSpec((NC, blk, 16), lambda i: (0, i, 0)),
            pl.BlockSpec((N_HEADS, ROW), lambda i: (0, 0)),
        ],
        out_specs=pl.BlockSpec((blk, ROW), lambda i: (i, 0)),
        out_shape=jax.ShapeDtypeStruct((N_NODES, ROW), jnp.float32),
    )(pwv, pz, bmat)


_BMAT = np.zeros((N_HEADS, ROW), dtype=np.float32)
for _h in range(N_HEADS):
    _BMAT[_h, _h * OUT_DIM:(_h + 1) * OUT_DIM] = 1.0
_BMAT.setflags(write=False)

_PERM = np.zeros((ROW,), dtype=np.int32)
for _g in range(N_HEADS // 2):
    for _d in range(OUT_DIM):
        _PERM[32 * _g + 2 * _d] = 32 * _g + _d
        _PERM[32 * _g + 2 * _d + 1] = 32 * _g + OUT_DIM + _d
_PERM.setflags(write=False)


def kernel(x, edge_index, Wq, Wk, Wv):
    perm = jnp.asarray(_PERM)
    q, k, v = _qkv(x, Wq[:, perm], Wk[:, perm], Wv)
    pwv, pz = _sc_attn(k, q, v, edge_index)
    out = _combine(pwv, pz, jnp.asarray(_BMAT))
    return out.reshape(N_NODES, N_HEADS, OUT_DIM)


# speedup vs baseline: 5.6438x; 1.0303x over previous
import functools

import jax
import jax.numpy as jnp
import numpy as np
from jax import lax
from jax.experimental import pallas as pl
from jax.experimental.pallas import tpu as pltpu
from jax.experimental.pallas import tpu_sc as plsc

N_NODES = 10000
N_EDGES = 320000
IN_DIM = 128
OUT_DIM = 16
N_HEADS = 8
ROW = OUT_DIM * N_HEADS
ZROW = ROW + 16

NC = 2
NS = 16
NW = NC * NS
EPW = N_EDGES // NW
CHUNK = 40
NCHUNK = EPW // CHUNK
RPT = N_NODES // NS


def _qkv_body(x_ref, wq_ref, wk_ref, wv_ref, pm_ref, q_ref, k_ref, v_ref):
    xb = x_ref[...]
    pm = pm_ref[...]
    wq = jnp.dot(wq_ref[...], pm, preferred_element_type=jnp.float32)
    wk = jnp.dot(wk_ref[...], pm, preferred_element_type=jnp.float32) * 0.25
    q_ref[...] = jnp.dot(
        xb, wq, preferred_element_type=jnp.float32).astype(jnp.bfloat16)
    k_ref[...] = jnp.dot(
        xb, wk, preferred_element_type=jnp.float32).astype(jnp.bfloat16)
    v_ref[...] = jnp.dot(xb, wv_ref[...], preferred_element_type=jnp.float32)


def _qkv(x, Wq, Wk, Wv):
    blk = 2000
    grid = (N_NODES // blk,)
    outh = jax.ShapeDtypeStruct((N_NODES, ROW), jnp.bfloat16)
    outf = jax.ShapeDtypeStruct((N_NODES, ROW), jnp.float32)
    return pl.pallas_call(
        _qkv_body,
        grid=grid,
        in_specs=[
            pl.BlockSpec((blk, IN_DIM), lambda i: (i, 0)),
            pl.BlockSpec((IN_DIM, ROW), lambda i: (0, 0)),
            pl.BlockSpec((IN_DIM, ROW), lambda i: (0, 0)),
            pl.BlockSpec((IN_DIM, ROW), lambda i: (0, 0)),
            pl.BlockSpec((ROW, ROW), lambda i: (0, 0)),
        ],
        out_specs=[
            pl.BlockSpec((blk, ROW), lambda i: (i, 0)),
            pl.BlockSpec((blk, ROW), lambda i: (i, 0)),
            pl.BlockSpec((blk, ROW), lambda i: (i, 0)),
        ],
        out_shape=[outh, outh, outf],
    )(x, Wq, Wk, Wv, jnp.asarray(_PMAT))


_sc_mesh = plsc.VectorSubcoreMesh(core_axis_name="c", subcore_axis_name="s")


@functools.partial(
    pl.kernel,
    mesh=_sc_mesh,
    compiler_params=pltpu.CompilerParams(use_tc_tiling_on_sc=False,
                                         needs_layout_passes=False),
    out_type=(jax.ShapeDtypeStruct((NC, N_NODES, ROW), jnp.float32),
              jax.ShapeDtypeStruct((NC, N_NODES, 16), jnp.float32)),
    scratch_types=[
        pltpu.VMEM((4, CHUNK), jnp.int32),
        pltpu.VMEM((4, CHUNK), jnp.int32),
        pltpu.VMEM((2, CHUNK, ROW), jnp.bfloat16),
        pltpu.VMEM((2, CHUNK, ROW), jnp.bfloat16),
        pltpu.VMEM((2, CHUNK, ROW), jnp.float32),
        pltpu.VMEM((2, CHUNK, 16), jnp.float32),
        pltpu.VMEM_SHARED((N_NODES, ROW), jnp.float32),
        pltpu.VMEM_SHARED((N_NODES, 16), jnp.float32),
        pltpu.SemaphoreType.DMA,
        pltpu.SemaphoreType.DMA,
        pltpu.SemaphoreType.DMA,
        pltpu.SemaphoreType.DMA,
        pltpu.SemaphoreType.DMA,
        pltpu.SemaphoreType.DMA,
        pltpu.SemaphoreType.DMA,
    ],
)
def _sc_attn(k_hbm, q_hbm, v_hbm, edge_hbm, outwv_hbm, outz_hbm,
             srcv, dstv, kr, qr, vr, zmsg, accwv, accz,
             sem1, sem2, sem3, semi0, semi1, semw, semz):
    cid = lax.axis_index("c")
    sid = lax.axis_index("s")
    wid = sid * NC + cid

    zeros16 = jnp.zeros((16,), jnp.float32)

    def zero_row(r, carry):
        for j in range(ROW // 16):
            vr[0, r, pl.ds(j * 16, 16)] = zeros16
        zmsg[0, r, pl.ds(0, 16)] = zeros16
        return carry

    lax.fori_loop(0, CHUNK, zero_row, 0)
    zc = []
    for k in range(RPT // CHUNK):
        base = sid * RPT + k * CHUNK
        zc.append(pltpu.async_copy(
            vr.at[0], accwv.at[pl.ds(base, CHUNK)], semw))
        zc.append(pltpu.async_copy(
            zmsg.at[0], accz.at[pl.ds(base, CHUNK)], semz))
    tail = RPT % CHUNK
    tbase = sid * RPT + (RPT // CHUNK) * CHUNK
    zc.append(pltpu.async_copy(
        vr.at[0, pl.ds(0, tail)], accwv.at[pl.ds(tbase, tail)], semw))
    zc.append(pltpu.async_copy(
        zmsg.at[0, pl.ds(0, tail)], accz.at[pl.ds(tbase, tail)], semz))
    for c in zc:
        c.wait()
    plsc.subcore_barrier()

    lanes = lax.iota(jnp.int32, 16)
    hmask = [lanes == h for h in range(N_HEADS)]
    hsel = [jnp.full((16,), h, jnp.int32) for h in range(N_HEADS)]

    def fetch_idx(i, bi, semi):
        base = wid * EPW + i * CHUNK
        pltpu.async_copy(edge_hbm.at[0, pl.ds(base, CHUNK)], srcv.at[bi], semi)
        pltpu.async_copy(edge_hbm.at[1, pl.ds(base, CHUNK)], dstv.at[bi], semi)

    def wait_idx(i, bi, semi):
        base = wid * EPW + i * CHUNK
        pltpu.make_async_copy(edge_hbm.at[0, pl.ds(base, CHUNK)],
                              srcv.at[bi], semi).wait()
        pltpu.make_async_copy(edge_hbm.at[1, pl.ds(base, CHUNK)],
                              dstv.at[bi], semi).wait()

    def fetch_rows(bi, b):
        pltpu.async_copy(k_hbm.at[srcv.at[bi]], kr.at[b], sem1)
        pltpu.async_copy(q_hbm.at[dstv.at[bi]], qr.at[b], sem2)
        pltpu.async_copy(v_hbm.at[srcv.at[bi]], vr.at[b], sem3)

    def drain_rows(bi, b):
        pltpu.make_async_copy(k_hbm.at[srcv.at[bi]], kr.at[b], sem1).wait()
        pltpu.make_async_copy(q_hbm.at[dstv.at[bi]], qr.at[b], sem2).wait()
        pltpu.make_async_copy(v_hbm.at[srcv.at[bi]], vr.at[b], sem3).wait()

    def start_scatter(bi, b):
        pltpu.async_copy(vr.at[b], accwv.at[dstv.at[bi]], semw, add=True)
        pltpu.async_copy(zmsg.at[b], accz.at[dstv.at[bi]], semz, add=True)

    def wait_scatter(bi, b):
        pltpu.make_async_copy(vr.at[b], accwv.at[dstv.at[bi]], semw).wait()
        pltpu.make_async_copy(zmsg.at[b], accz.at[dstv.at[bi]], semz).wait()

    def compute(b, e):
        zvec = jnp.zeros((16,), jnp.float32)
        for g in range(N_HEADS // 2):
            kv2 = kr[b, e, pl.ds(32 * g, 32)]
            qv2 = qr[b, e, pl.ds(32 * g, 32)]
            ka, kb = plsc.unpack(kv2, format=plsc.PackFormat.INTERLEAVED)
            qa, qb = plsc.unpack(qv2, format=plsc.PackFormat.INTERLEAVED)
            for h, kv, qv in ((2 * g, ka, qa), (2 * g + 1, kb, qb)):
                s = jnp.clip(jnp.sum(kv * qv), -5.0, 5.0)
                zvec = jnp.where(hmask[h], s, zvec)
        ez = jnp.exp(zvec)
        zmsg[b, e, pl.ds(0, 16)] = ez
        for h in range(N_HEADS):
            sv = ez.at[hsel[h]].get(mode="promise_in_bounds")
            vr[b, e, pl.ds(h * 16, 16)] = vr[b, e, pl.ds(h * 16, 16)] * sv

    def half_body(i, b, semi_here, semi_next):
        @pl.when(i + 2 < NCHUNK)
        def _():
            fetch_idx(i + 2, lax.rem(i + 2, 4), semi_here)

        @pl.when(i + 1 < NCHUNK)
        def _():
            wait_idx(i + 1, lax.rem(i + 1, 4), semi_next)

            @pl.when(i >= 1)
            def _():
                wait_scatter(lax.rem(i + 3, 4), 1 - b)

            fetch_rows(lax.rem(i + 1, 4), 1 - b)

        drain_rows(lax.rem(i, 4), b)

        @plsc.parallel_loop(0, CHUNK, unroll=4)
        def edge_body(e):
            compute(b, e)

        start_scatter(lax.rem(i, 4), b)

    fetch_idx(0, 0, semi0)
    fetch_idx(1, 1, semi1)
    wait_idx(0, 0, semi0)
    fetch_rows(0, 0)

    def chunk_pair(j, carry):
        i0 = 2 * j
        half_body(i0, 0, semi0, semi1)
        half_body(i0 + 1, 1, semi1, semi0)
        return carry

    lax.fori_loop(0, NCHUNK // 2, chunk_pair, 0)
    wait_scatter((NCHUNK - 1) % 4, (NCHUNK - 1) % 2)
    plsc.subcore_barrier()

    pltpu.sync_copy(accwv.at[pl.ds(sid * RPT, RPT)],
                    outwv_hbm.at[cid, pl.ds(sid * RPT, RPT)])
    pltpu.sync_copy(accz.at[pl.ds(sid * RPT, RPT)],
                    outz_hbm.at[cid, pl.ds(sid * RPT, RPT)])


def _combine_body(pwv_ref, pz_ref, b_ref, o_ref):
    wv = pwv_ref[0] + pwv_ref[1]
    z = pz_ref[0, :, :N_HEADS] + pz_ref[1, :, :N_HEADS]
    zb = jnp.dot(z, b_ref[...], preferred_element_type=jnp.float32)
    o_ref[...] = wv / (zb + 1e-6)


def _combine(pwv, pz, bmat):
    blk = 2000
    return pl.pallas_call(
        _combine_body,
        grid=(N_NODES // blk,),
        in_specs=[
            pl.BlockSpec((NC, blk, ROW), lambda i: (0, i, 0)),
            pl.BlockSpec((NC, blk, 16), lambda i: (0, i, 0)),
            pl.BlockSpec((N_HEADS, ROW), lambda i: (0, 0)),
        ],
        out_specs=pl.BlockSpec((blk, ROW), lambda i: (i, 0)),
        out_shape=jax.ShapeDtypeStruct((N_NODES, ROW), jnp.float32),
    )(pwv, pz, bmat)


_BMAT = np.zeros((N_HEADS, ROW), dtype=np.float32)
for _h in range(N_HEADS):
    _BMAT[_h, _h * OUT_DIM:(_h + 1) * OUT_DIM] = 1.0
_BMAT.setflags(write=False)

_PERM = np.zeros((ROW,), dtype=np.int32)
for _g in range(N_HEADS // 2):
    for _d in range(OUT_DIM):
        _PERM[32 * _g + 2 * _d] = 32 * _g + _d
        _PERM[32 * _g + 2 * _d + 1] = 32 * _g + OUT_DIM + _d
_PERM.setflags(write=False)
_PMAT = np.zeros((ROW, ROW), dtype=np.float32)
_PMAT[_PERM, np.arange(ROW)] = 1.0
_PMAT.setflags(write=False)


def kernel(x, edge_index, Wq, Wk, Wv):
    q, k, v = _qkv(x, Wq, Wk, Wv)
    pwv, pz = _sc_attn(k, q, v, edge_index)
    out = _combine(pwv, pz, jnp.asarray(_BMAT))
    return out.reshape(N_NODES, N_HEADS, OUT_DIM)
